# baseline (speedup 1.0000x reference)
import functools

import jax
import jax.numpy as jnp
from jax import lax
from jax.experimental import pallas as pl
from jax.experimental.pallas import tpu as pltpu
from jax.experimental.pallas import tpu_sc as plsc

_NUM_EMB = 1000000
_DIM = 32
_BATCH = 4096
_SEQ = 200
_NBT = _BATCH // 128


_CCH = 8192
_UG = (_NUM_EMB + _CCH - 1) // _CCH
_PROWS = _UG * (_CCH // 4)


def _untile_body(wt_ref, out_ref):
    for p in range(4):
        x = wt_ref[:, p * (_CCH // 4):(p + 1) * (_CCH // 4)]
        out_ref[:, p * _DIM:(p + 1) * _DIM] = jnp.transpose(x, (1, 0))


_untile = pl.pallas_call(
    _untile_body,
    grid=(_UG,),
    in_specs=[pl.BlockSpec((_DIM, _CCH), lambda j: (0, j))],
    out_specs=pl.BlockSpec((_CCH // 4, 128), lambda j: (j, 0)),
    out_shape=jax.ShapeDtypeStruct((_PROWS, 128), jnp.float32),
)


_info = plsc.get_sparse_core_info()
_NC, _NS = _info.num_cores, _info.num_subcores
_NW = _NC * _NS
_GS, _GBT = 8, 4
_SPG = _SEQ // _GS
_BTPG = _NBT // _GBT
_UNITS = _SPG * _BTPG
_NVEC = _SPG * _BTPG * 128 // 16

_mesh = plsc.VectorSubcoreMesh(core_axis_name="c", subcore_axis_name="s")


@functools.partial(
    pl.kernel,
    mesh=_mesh,
    out_type=jax.ShapeDtypeStruct((_SEQ, _NBT, 128, _DIM), jnp.float32),
    scratch_types=[
        pltpu.VMEM((_SPG, _BTPG, 128), jnp.int32),
        pltpu.VMEM((128, _DIM), jnp.float32),
        pltpu.VMEM((128, _DIM), jnp.float32),
        pltpu.SemaphoreType.DMA,
        pltpu.SemaphoreType.DMA,
        pltpu.SemaphoreType.DMA,
        pltpu.SemaphoreType.DMA,
    ],
    compiler_params=pltpu.CompilerParams(
        use_tc_tiling_on_sc=False, needs_layout_passes=False
    ),
)
def _gather_sc(table_hbm, idx_hbm, out_hbm, idx_v, rows0, rows1,
               semg0, semg1, sems0, sems1):
    wid = lax.axis_index("s") * _NC + lax.axis_index("c")
    gs = wid // _GBT
    gbt = wid % _GBT
    rows = (rows0, rows1)
    semg = (semg0, semg1)
    sems = (sems0, sems1)

    pltpu.sync_copy(
        idx_hbm.at[pl.ds(gs * _SPG, _SPG), pl.ds(gbt * _BTPG, _BTPG)], idx_v
    )

    @pl.loop(0, _NVEC, step=1)
    def _xform(i):
        si = i // (_BTPG * 8)
        bi = (i // 8) % _BTPG
        v = i % 8
        r = idx_v[si, bi, pl.ds(v * 16, 16)]
        m = ((r & jnp.int32(~8191)) | ((r & jnp.int32(2047)) << 2)
             | ((r >> 11) & jnp.int32(3)))
        idx_v[si, bi, pl.ds(v * 16, 16)] = m

    def fire_g(u, buf):
        pltpu.async_copy(
            table_hbm.at[idx_v.at[u // _BTPG, u % _BTPG]], rows[buf], semg[buf]
        )

    def drain_g(buf):
        pltpu.make_async_copy(
            table_hbm.at[pl.ds(0, 128)], rows[buf], semg[buf]
        ).wait()

    def fire_s(u, buf):
        s = gs * _SPG + u // _BTPG
        bt = gbt * _BTPG + u % _BTPG
        pltpu.async_copy(rows[buf], out_hbm.at[s, bt], sems[buf])

    def drain_s(buf):
        pltpu.make_async_copy(rows[buf], out_hbm.at[0, 0], sems[buf]).wait()

    fire_g(0, 0)
    fire_g(1, 1)

    @pl.loop(0, _UNITS - 2, step=2)
    def _steady(u0):
        for d_ in range(2):
            u = u0 + d_
            buf = d_ % 2
            drain_g(buf)
            fire_s(u, buf)
            drain_s(buf)
            fire_g(u + 2, buf)

    for u, buf in ((_UNITS - 2, 0), (_UNITS - 1, 1)):
        drain_g(buf)
        fire_s(u, buf)
        drain_s(buf)


_XBT = 8


def _xpose_body(in_ref, out_ref):
    for k in range(_XBT):
        x = in_ref[k * 32:(k + 1) * 32, :]
        cols = [
            jnp.transpose(x[:, p * _DIM:(p + 1) * _DIM], (1, 0))
            for p in range(4)
        ]
        y = jnp.stack(cols, axis=-1).reshape(_DIM, 128)
        out_ref[0, :, k * 128:(k + 1) * 128] = y


_xpose = pl.pallas_call(
    _xpose_body,
    grid=(_SEQ, _NBT // _XBT),
    in_specs=[pl.BlockSpec((_XBT * 32, 128),
                           lambda s, g: (s * (_NBT // _XBT) + g, 0))],
    out_specs=pl.BlockSpec((1, _DIM, _XBT * 128), lambda s, g: (s, 0, g)),
    out_shape=jax.ShapeDtypeStruct((_SEQ, _DIM, _BATCH), jnp.float32),
)


def kernel(input_ids, weight):
    table_lin = _untile(weight.T).reshape(_PROWS * 4, _DIM)
    idx3 = input_ids.T.reshape(_SEQ, _NBT, 128)
    inter = _gather_sc(table_lin, idx3)
    inter2 = inter.reshape(_SEQ * _NBT * 32, 128)
    out = _xpose(inter2)
    return out.transpose(2, 0, 1)


# speedup vs baseline: 8.0028x; 8.0028x over previous
import functools

import jax
import jax.numpy as jnp
from jax import lax
from jax.experimental import pallas as pl
from jax.experimental.pallas import tpu as pltpu
from jax.experimental.pallas import tpu_sc as plsc

_NUM_EMB = 1000000
_DIM = 32
_BATCH = 4096
_SEQ = 200
_B = _BATCH * _SEQ


_CCH = 8192
_UG = (_NUM_EMB + _CCH - 1) // _CCH
_PROWS = _UG * (_CCH // 4)


def _untile_body(wt_ref, out_ref):
    for p in range(4):
        x = wt_ref[:, p * (_CCH // 4):(p + 1) * (_CCH // 4)]
        out_ref[:, p * _DIM:(p + 1) * _DIM] = jnp.transpose(x, (1, 0))


_untile = pl.pallas_call(
    _untile_body,
    grid=(_UG,),
    in_specs=[pl.BlockSpec((_DIM, _CCH), lambda j: (0, j))],
    out_specs=pl.BlockSpec((_CCH // 4, 128), lambda j: (j, 0)),
    out_shape=jax.ShapeDtypeStruct((_PROWS, 128), jnp.float32),
)


_info = plsc.get_sparse_core_info()
_NC, _NS = _info.num_cores, _info.num_subcores
_NW = _NC * _NS
_BPW = _B // _NW
_CH = 1280
_NCH = _BPW // _CH
_NVEC = _BPW // 16

_mesh = plsc.VectorSubcoreMesh(core_axis_name="c", subcore_axis_name="s")


@functools.partial(
    pl.kernel,
    mesh=_mesh,
    out_type=jax.ShapeDtypeStruct((_B, _DIM), jnp.float32),
    scratch_types=[
        pltpu.VMEM((_NCH, _CH), jnp.int32),
        pltpu.VMEM((_CH, _DIM), jnp.float32),
        pltpu.VMEM((_CH, _DIM), jnp.float32),
        pltpu.SemaphoreType.DMA,
        pltpu.SemaphoreType.DMA,
        pltpu.SemaphoreType.DMA,
        pltpu.SemaphoreType.DMA,
    ],
    compiler_params=pltpu.CompilerParams(
        use_tc_tiling_on_sc=False, needs_layout_passes=False
    ),
)
def _gather_sc(table_hbm, idx_hbm, out_hbm, idx_v, rows0, rows1,
               semg0, semg1, sems0, sems1):
    wid = lax.axis_index("s") * _NC + lax.axis_index("c")
    base = wid * _BPW
    rows = (rows0, rows1)
    semg = (semg0, semg1)
    sems = (sems0, sems1)

    pltpu.sync_copy(idx_hbm.at[wid], idx_v)

    @pl.loop(0, _NVEC, step=1)
    def _xform(i):
        si = i // (_CH // 16)
        v = i % (_CH // 16)
        r = idx_v[si, pl.ds(v * 16, 16)]
        m = ((r & jnp.int32(~8191)) | ((r & jnp.int32(2047)) << 2)
             | ((r >> 11) & jnp.int32(3)))
        idx_v[si, pl.ds(v * 16, 16)] = m

    def fire_g(t, buf):
        pltpu.async_copy(table_hbm.at[idx_v.at[t]], rows[buf], semg[buf])

    def drain_g(buf):
        pltpu.make_async_copy(
            table_hbm.at[pl.ds(0, _CH)], rows[buf], semg[buf]
        ).wait()

    def fire_s(t, buf):
        pltpu.async_copy(
            rows[buf], out_hbm.at[pl.ds(base + t * _CH, _CH)], sems[buf]
        )

    def drain_s(buf):
        pltpu.make_async_copy(
            rows[buf], out_hbm.at[pl.ds(0, _CH)], sems[buf]
        ).wait()

    fire_g(0, 0)
    drain_g(0)
    fire_s(0, 0)
    fire_g(1, 1)

    @pl.loop(1, _NCH - 1, step=2)
    def _steady(t0):
        for d_ in range(2):
            t = t0 + d_
            buf = (1 + d_) % 2
            nbuf = 1 - buf
            drain_g(buf)
            fire_s(t, buf)
            drain_s(nbuf)
            fire_g(t + 1, nbuf)

    drain_g(1)
    fire_s(_NCH - 1, 1)
    drain_s(0)
    drain_s(1)


def kernel(input_ids, weight):
    table_lin = _untile(weight.T).reshape(_PROWS * 4, _DIM)
    idx = input_ids.reshape(_NW, _NCH, _CH)
    out = _gather_sc(table_lin, idx)
    return out.reshape(_BATCH, _SEQ, _DIM)
